# Initial kernel scaffold; baseline (speedup 1.0000x reference)
import jax
import jax.numpy as jnp
from jax.experimental import pallas as pl


def kernel(hidden_states, W_router, W_gate, W_up, W_down, expert_amplitudes, expert_scale, Wg_sh, Wu_sh, Wd_sh):
    raise NotImplementedError("write your pallas kernel here")


# speedup vs baseline: 2.9425x; 2.9425x over previous
import functools

import jax
import jax.numpy as jnp
from jax.experimental import pallas as pl
from jax.experimental.pallas import tpu as pltpu

B, S, H = 2, 4096, 1024
E, K = 64, 2
NB = 8
INTER = 256
N_SHARED = 2
SH_INTER = INTER * N_SHARED
EPS = 1e-8

TM = 512


def _moe_body(x_ref, wr_ref, wg_ref, wu_ref, wd_ref, amp_ref, scale_ref,
              wgsh_ref, wush_ref, wdsh_ref, o_ref):
    x = x_ref[...]

    logits = jnp.dot(x, wr_ref[...], preferred_element_type=jnp.float32)
    m = jnp.max(logits, axis=-1, keepdims=True)
    ex = jnp.exp(logits - m)
    probs = ex / jnp.sum(ex, axis=-1, keepdims=True)

    e_iota = jax.lax.broadcasted_iota(jnp.int32, probs.shape, 1)
    v1 = jnp.max(probs, axis=-1, keepdims=True)
    i1 = jnp.min(jnp.where(probs == v1, e_iota, E), axis=-1, keepdims=True)
    mask1 = e_iota == i1
    rest = jnp.where(mask1, -jnp.inf, probs)
    v2 = jnp.max(rest, axis=-1, keepdims=True)
    i2 = jnp.min(jnp.where(rest == v2, e_iota, E), axis=-1, keepdims=True)
    mask2 = e_iota == i2
    denom = v1 + v2 + EPS
    g = jnp.where(mask1 | mask2, probs, 0.0) / denom

    a0 = amp_ref[0]
    a1 = amp_ref[1]
    ap = a0 * a0 + a1 * a1
    ap = ap / (jnp.sum(ap, axis=-1, keepdims=True) + EPS)
    amp_scaled = ap * scale_ref[...]

    c = jnp.dot(g, amp_scaled, preferred_element_type=jnp.float32)

    gate = jnp.dot(x, wg_ref[...], preferred_element_type=jnp.float32)
    up = jnp.dot(x, wu_ref[...], preferred_element_type=jnp.float32)
    basis = (gate * jax.nn.sigmoid(gate)) * up

    combined = c[:, 0:1] * basis[:, 0:INTER]
    for b in range(1, NB):
        combined = combined + c[:, b:b + 1] * basis[:, b * INTER:(b + 1) * INTER]

    sg = jnp.dot(x, wgsh_ref[...], preferred_element_type=jnp.float32)
    su = jnp.dot(x, wush_ref[...], preferred_element_type=jnp.float32)
    sh = (sg * jax.nn.sigmoid(sg)) * su

    o_ref[...] = (
        jnp.dot(combined, wd_ref[...], preferred_element_type=jnp.float32)
        + jnp.dot(sh, wdsh_ref[...], preferred_element_type=jnp.float32)
    )


@jax.jit
def _moe_fused(x, W_router, W_gate, W_up, W_down, amp_t, scale_c,
               Wg_sh, Wu_sh, Wd_sh):
    T = x.shape[0]
    grid = (T // TM,)

    def tile(i):
        return (i, 0)

    def whole(i):
        return (0, 0)

    return pl.pallas_call(
        _moe_body,
        grid=grid,
        in_specs=[
            pl.BlockSpec((TM, H), tile),
            pl.BlockSpec((H, E), whole),
            pl.BlockSpec((H, NB * INTER), whole),
            pl.BlockSpec((H, NB * INTER), whole),
            pl.BlockSpec((INTER, H), whole),
            pl.BlockSpec((2, E, NB), lambda i: (0, 0, 0)),
            pl.BlockSpec((E, 1), whole),
            pl.BlockSpec((H, SH_INTER), whole),
            pl.BlockSpec((H, SH_INTER), whole),
            pl.BlockSpec((SH_INTER, H), whole),
        ],
        out_specs=pl.BlockSpec((TM, H), tile),
        out_shape=jax.ShapeDtypeStruct((T, H), jnp.float32),
    )(x, W_router, W_gate, W_up, W_down, amp_t, scale_c, Wg_sh, Wu_sh, Wd_sh)


def kernel(hidden_states, W_router, W_gate, W_up, W_down, expert_amplitudes,
           expert_scale, Wg_sh, Wu_sh, Wd_sh):
    T = B * S
    x = hidden_states.reshape(T, H)
    amp_t = expert_amplitudes.transpose(2, 0, 1)
    scale_c = expert_scale.reshape(E, 1)
    out = _moe_fused(x, W_router, W_gate, W_up, W_down, amp_t, scale_c,
                     Wg_sh, Wu_sh, Wd_sh)
    return out.reshape(B, S, H)
